# Initial kernel scaffold; baseline (speedup 1.0000x reference)
import functools

import jax
import jax.numpy as jnp
from jax import lax
from jax.experimental import pallas as pl
from jax.experimental.pallas import tpu as pltpu
from jax.experimental.pallas import tpu_sc as plsc

N = 320000
B = 10000
H = 256

NC = 2
NS = 16
NW = NC * NS
PTS_PER_TILE = N // NW
CHUNK = 80
NCHUNK = PTS_PER_TILE // CHUNK
BPAD = 10240
RPT = BPAD // NS
R_MLP = 2560


def _sc_gather(table, idx3):
    mesh = plsc.VectorSubcoreMesh(core_axis_name="c", subcore_axis_name="s")

    @functools.partial(
        pl.kernel,
        mesh=mesh,
        out_type=jax.ShapeDtypeStruct((N, 4), jnp.float32),
        scratch_types=[
            pltpu.VMEM((NCHUNK, CHUNK), jnp.int32),
            pltpu.VMEM((PTS_PER_TILE, 4), jnp.float32),
            pltpu.SemaphoreType.DMA,
        ],
    )
    def k(table_hbm, idx_hbm, out_hbm, idx_v, rows_v, sem):
        cid = lax.axis_index("c")
        sid = lax.axis_index("s")
        wid = sid * NC + cid
        pltpu.sync_copy(idx_hbm.at[wid], idx_v)

        def body(j, carry):
            pltpu.async_copy(
                table_hbm.at[idx_v.at[j]],
                rows_v.at[pl.ds(j * CHUNK, CHUNK)],
                sem,
            ).wait()
            return carry

        lax.fori_loop(0, NCHUNK, body, 0)
        pltpu.sync_copy(rows_v, out_hbm.at[pl.ds(wid * PTS_PER_TILE, PTS_PER_TILE)])

    return k(table, idx3)


def _sc_scatter(vals3, idx3, zeros):
    mesh = plsc.VectorSubcoreMesh(core_axis_name="c", subcore_axis_name="s")

    @functools.partial(
        pl.kernel,
        mesh=mesh,
        out_type=jax.ShapeDtypeStruct((NC, BPAD, 4), jnp.float32),
        scratch_types=[
            pltpu.VMEM((NCHUNK, CHUNK), jnp.int32),
            pltpu.VMEM((PTS_PER_TILE, 4), jnp.float32),
            pltpu.VMEM((RPT, 4), jnp.float32),
            pltpu.VMEM_SHARED((BPAD, 4), jnp.float32),
        ],
    )
    def k(vals_hbm, idx_hbm, zeros_hbm, out_hbm, idx_v, vals_v, obuf_v, acc_sh):
        cid = lax.axis_index("c")
        sid = lax.axis_index("s")
        wid = sid * NC + cid
        pltpu.sync_copy(zeros_hbm.at[pl.ds(sid * RPT, RPT)], obuf_v)
        pltpu.sync_copy(obuf_v, acc_sh.at[pl.ds(sid * RPT, RPT)])
        pltpu.sync_copy(idx_hbm.at[wid], idx_v)
        pltpu.sync_copy(vals_hbm.at[wid], vals_v)
        plsc.subcore_barrier()

        def body(j, carry):
            pltpu.sync_copy(
                vals_v.at[pl.ds(j * CHUNK, CHUNK)],
                acc_sh.at[idx_v.at[j]],
                add=True,
            )
            return carry

        lax.fori_loop(0, NCHUNK, body, 0)
        plsc.subcore_barrier()
        pltpu.sync_copy(acc_sh.at[pl.ds(sid * RPT, RPT)], obuf_v)
        pltpu.sync_copy(obuf_v, out_hbm.at[cid, pl.ds(sid * RPT, RPT)])

    return k(vals3, idx3, zeros)


def _silu(x):
    return x / (1.0 + jnp.exp(-x))


def _tc_mlp(pts, g, W0, b0, W1, b1, w2t, b2):
    def body(pts_ref, g_ref, w0_ref, b0_ref, w1_ref, b1_ref, w2_ref, b2_ref, out_ref):
        pts_b = pts_ref[...]
        g_b = g_ref[...]
        diff_t = jnp.sign(pts_b[:, 0:1] - g_b[:, 0:1])
        dp = pts_b[:, 1:4] - g_b[:, 1:4]
        r2 = jnp.sum(dp * dp, axis=1, keepdims=True)
        h = _silu(diff_t * w0_ref[0:1, :] + r2 * w0_ref[1:2, :] + b0_ref[...])
        h = _silu(
            jnp.dot(h, w1_ref[...], preferred_element_type=jnp.float32) + b1_ref[...]
        )
        w = jnp.sum(h * w2_ref[...], axis=1, keepdims=True) + b2_ref[...]
        inv = 1.0 / jnp.maximum(jnp.sqrt(r2), 1e-12)
        vals = dp * (w * inv)
        out_ref[...] = jnp.concatenate(
            [vals, jnp.zeros((vals.shape[0], 1), jnp.float32)], axis=1
        )

    grid = (N // R_MLP,)
    return pl.pallas_call(
        body,
        grid=grid,
        in_specs=[
            pl.BlockSpec((R_MLP, 4), lambda i: (i, 0)),
            pl.BlockSpec((R_MLP, 4), lambda i: (i, 0)),
            pl.BlockSpec((2, H), lambda i: (0, 0)),
            pl.BlockSpec((1, H), lambda i: (0, 0)),
            pl.BlockSpec((H, H), lambda i: (0, 0)),
            pl.BlockSpec((1, H), lambda i: (0, 0)),
            pl.BlockSpec((1, H), lambda i: (0, 0)),
            pl.BlockSpec((1, 1), lambda i: (0, 0)),
        ],
        out_specs=pl.BlockSpec((R_MLP, 4), lambda i: (i, 0)),
        out_shape=jax.ShapeDtypeStruct((N, 4), jnp.float32),
    )(pts, g, W0, b0, W1, b1, w2t, b2)


def _tc_norm(partials):
    def body(p_ref, out_ref):
        p = p_ref[0] + p_ref[1]
        n2 = jnp.sum(p * p, axis=1, keepdims=True)
        out_ref[...] = p * (1.0 / jnp.maximum(jnp.sqrt(n2), 1e-12))

    return pl.pallas_call(
        body,
        out_shape=jax.ShapeDtypeStruct((BPAD, 4), jnp.float32),
    )(partials)


def kernel(t, pos, poi_t, poi_pos, batch, W0, b0, W1, b1, W2, b2):
    idx = batch.astype(jnp.int32)
    pts = jnp.concatenate([t[:, None], pos], axis=1)
    table = jnp.concatenate([poi_t[:, None], poi_pos], axis=1)
    idx3 = idx.reshape(NW, NCHUNK, CHUNK)
    g = _sc_gather(table, idx3)
    vals = _tc_mlp(
        pts,
        g,
        W0,
        b0.reshape(1, H),
        W1,
        b1.reshape(1, H),
        W2.reshape(1, H),
        b2.reshape(1, 1),
    )
    partials = _sc_scatter(
        vals.reshape(NW, PTS_PER_TILE, 4), idx3, jnp.zeros((BPAD, 4), jnp.float32)
    )
    out = _tc_norm(partials)
    return out[:B, :3]


# speedup vs baseline: 5.0659x; 5.0659x over previous
import functools

import jax
import jax.numpy as jnp
from jax import lax
from jax.experimental import pallas as pl
from jax.experimental.pallas import tpu as pltpu
from jax.experimental.pallas import tpu_sc as plsc

N = 320000
B = 10000
H = 256

NC = 2
NS = 16
NW = NC * NS
PTS_PER_TILE = N // NW
CHUNK = 80
NCHUNK = PTS_PER_TILE // CHUNK
BPAD = 10240
RPT = BPAD // NS
R_MLP = 2560
ROWW = 8


def _sc_gather(table, idx3):
    mesh = plsc.VectorSubcoreMesh(core_axis_name="c", subcore_axis_name="s")

    @functools.partial(
        pl.kernel,
        mesh=mesh,
        out_type=jax.ShapeDtypeStruct((N, ROWW), jnp.float32),
        scratch_types=[
            pltpu.VMEM((NCHUNK, CHUNK), jnp.int32),
            pltpu.VMEM((PTS_PER_TILE, ROWW), jnp.float32),
            pltpu.SemaphoreType.DMA,
        ],
        compiler_params=pltpu.CompilerParams(use_tc_tiling_on_sc=False),
    )
    def k(table_hbm, idx_hbm, out_hbm, idx_v, rows_v, sem):
        cid = lax.axis_index("c")
        sid = lax.axis_index("s")
        wid = sid * NC + cid
        pltpu.sync_copy(idx_hbm.at[wid], idx_v)

        def body(j, carry):
            pltpu.async_copy(
                table_hbm.at[idx_v.at[j]],
                rows_v.at[pl.ds(j * CHUNK, CHUNK)],
                sem,
            ).wait()
            return carry

        lax.fori_loop(0, NCHUNK, body, 0)
        pltpu.sync_copy(rows_v, out_hbm.at[pl.ds(wid * PTS_PER_TILE, PTS_PER_TILE)])

    return k(table, idx3)


def _sc_scatter(vals3, idx3, zeros):
    mesh = plsc.VectorSubcoreMesh(core_axis_name="c", subcore_axis_name="s")

    @functools.partial(
        pl.kernel,
        mesh=mesh,
        out_type=jax.ShapeDtypeStruct((NC, BPAD, ROWW), jnp.float32),
        scratch_types=[
            pltpu.VMEM((NCHUNK, CHUNK), jnp.int32),
            pltpu.VMEM((PTS_PER_TILE, ROWW), jnp.float32),
            pltpu.VMEM((RPT, ROWW), jnp.float32),
            pltpu.VMEM_SHARED((BPAD, ROWW), jnp.float32),
        ],
        compiler_params=pltpu.CompilerParams(use_tc_tiling_on_sc=False),
    )
    def k(vals_hbm, idx_hbm, zeros_hbm, out_hbm, idx_v, vals_v, obuf_v, acc_sh):
        cid = lax.axis_index("c")
        sid = lax.axis_index("s")
        wid = sid * NC + cid
        pltpu.sync_copy(zeros_hbm.at[pl.ds(sid * RPT, RPT)], obuf_v)
        pltpu.sync_copy(obuf_v, acc_sh.at[pl.ds(sid * RPT, RPT)])
        pltpu.sync_copy(idx_hbm.at[wid], idx_v)
        pltpu.sync_copy(vals_hbm.at[wid], vals_v)
        plsc.subcore_barrier()

        def body(j, carry):
            pltpu.sync_copy(
                vals_v.at[pl.ds(j * CHUNK, CHUNK)],
                acc_sh.at[idx_v.at[j]],
                add=True,
            )
            return carry

        lax.fori_loop(0, NCHUNK, body, 0)
        plsc.subcore_barrier()
        pltpu.sync_copy(acc_sh.at[pl.ds(sid * RPT, RPT)], obuf_v)
        pltpu.sync_copy(obuf_v, out_hbm.at[cid, pl.ds(sid * RPT, RPT)])

    return k(vals3, idx3, zeros)


def _silu(x):
    return x / (1.0 + jnp.exp(-x))


def _tc_mlp(pts, g, W0, b0, W1, b1, w2t, b2):
    def body(pts_ref, g_ref, w0_ref, b0_ref, w1_ref, b1_ref, w2_ref, b2_ref, out_ref):
        pts_b = pts_ref[...]
        g_b = g_ref[...]
        diff_t = jnp.sign(pts_b[:, 0:1] - g_b[:, 0:1])
        dp = pts_b[:, 1:4] - g_b[:, 1:4]
        r2 = jnp.sum(dp * dp, axis=1, keepdims=True)
        h = _silu(diff_t * w0_ref[0:1, :] + r2 * w0_ref[1:2, :] + b0_ref[...])
        h = _silu(
            jnp.dot(h, w1_ref[...], preferred_element_type=jnp.float32) + b1_ref[...]
        )
        w = jnp.sum(h * w2_ref[...], axis=1, keepdims=True) + b2_ref[...]
        inv = 1.0 / jnp.maximum(jnp.sqrt(r2), 1e-12)
        vals = dp * (w * inv)
        out_ref[...] = jnp.concatenate(
            [vals, jnp.zeros((vals.shape[0], ROWW - 3), jnp.float32)], axis=1
        )

    grid = (N // R_MLP,)
    return pl.pallas_call(
        body,
        grid=grid,
        in_specs=[
            pl.BlockSpec((R_MLP, 4), lambda i: (i, 0)),
            pl.BlockSpec((R_MLP, ROWW), lambda i: (i, 0)),
            pl.BlockSpec((2, H), lambda i: (0, 0)),
            pl.BlockSpec((1, H), lambda i: (0, 0)),
            pl.BlockSpec((H, H), lambda i: (0, 0)),
            pl.BlockSpec((1, H), lambda i: (0, 0)),
            pl.BlockSpec((1, H), lambda i: (0, 0)),
            pl.BlockSpec((1, 1), lambda i: (0, 0)),
        ],
        out_specs=pl.BlockSpec((R_MLP, ROWW), lambda i: (i, 0)),
        out_shape=jax.ShapeDtypeStruct((N, ROWW), jnp.float32),
    )(pts, g, W0, b0, W1, b1, w2t, b2)


def _tc_norm(partials):
    def body(p_ref, out_ref):
        p = p_ref[0] + p_ref[1]
        n2 = jnp.sum(p * p, axis=1, keepdims=True)
        out_ref[...] = p * (1.0 / jnp.maximum(jnp.sqrt(n2), 1e-12))

    return pl.pallas_call(
        body,
        out_shape=jax.ShapeDtypeStruct((BPAD, ROWW), jnp.float32),
    )(partials)


def kernel(t, pos, poi_t, poi_pos, batch, W0, b0, W1, b1, W2, b2):
    idx = batch.astype(jnp.int32)
    pts = jnp.concatenate([t[:, None], pos], axis=1)
    table = jnp.concatenate(
        [poi_t[:, None], poi_pos, jnp.zeros((B, ROWW - 4), jnp.float32)], axis=1
    )
    idx3 = idx.reshape(NW, NCHUNK, CHUNK)
    g = _sc_gather(table, idx3)
    vals = _tc_mlp(
        pts,
        g,
        W0,
        b0.reshape(1, H),
        W1,
        b1.reshape(1, H),
        W2.reshape(1, H),
        b2.reshape(1, 1),
    )
    partials = _sc_scatter(
        vals.reshape(NW, PTS_PER_TILE, ROWW), idx3, jnp.zeros((BPAD, ROWW), jnp.float32)
    )
    out = _tc_norm(partials)
    return out[:B, :3]
